# baseline (speedup 1.0000x reference)
import functools

import jax
import jax.numpy as jnp
from jax.experimental import pallas as pl
from jax.experimental.pallas import tpu as pltpu

_K = 1024
_C = 64
_COMMIT = 0.25

_TILE = 1024
_NCHUNK = 4


def _vq_tile_kernel(x_ref, e_ref, enc_ref, quant_ref, loss_ref, perp_ref,
                    esq_acc, colsum_acc, loss_acc, *, n_tok, n_steps):
    i = pl.program_id(0)
    e = e_ref[...]

    @pl.when(i == 0)
    def _init():
        esq_acc[...] = jnp.sum(e * e, axis=1, keepdims=True).reshape(1, _K)
        colsum_acc[...] = jnp.zeros_like(colsum_acc)
        loss_acc[...] = jnp.zeros_like(loss_acc)

    esq = esq_acc[...]
    xb = x_ref[0]
    xt = xb.T
    cs = _TILE // _NCHUNK
    colsums = []
    losssums = []
    for ci in range(_NCHUNK):
        x = xt[ci * cs:(ci + 1) * cs, :]

        xsq = jnp.sum(x * x, axis=1, keepdims=True)
        mm = jnp.dot(x, e.T, preferred_element_type=jnp.float32)
        d = (xsq + esq) - 2.0 * mm

        dmin = jnp.min(d, axis=1, keepdims=True)
        iota = jax.lax.broadcasted_iota(jnp.int32, (cs, _K), 1)
        idx = jnp.min(jnp.where(d == dmin, iota, _K), axis=1, keepdims=True)

        onehot = jnp.where(iota == idx, 1.0, 0.0)
        enc_ref[pl.ds(ci * cs, cs), :] = onehot

        quant = jnp.dot(onehot.astype(jnp.bfloat16), e.astype(jnp.bfloat16),
                        preferred_element_type=jnp.float32)
        quant_ref[0, :, pl.ds(ci * cs, cs)] = (x + (quant - x)).T

        colsums.append(jnp.sum(onehot, axis=0, keepdims=True))
        r = quant - x
        losssums.append(jnp.sum(r * r, axis=0, keepdims=True))

    colsum_acc[...] += sum(colsums)
    loss_acc[...] += sum(losssums)

    @pl.when(i == n_steps - 1)
    def _finalize():
        mse = jnp.sum(loss_acc[...]) / (n_tok * _C)
        loss_ref[...] = jnp.broadcast_to(mse + _COMMIT * mse, (1, 1))
        probs = colsum_acc[...] / n_tok
        ent = jnp.sum(probs * jnp.log(probs + 1e-10))
        perp_ref[...] = jnp.broadcast_to(jnp.exp(-ent), (1, 1))


@jax.jit
def kernel(inputs, embedding):
    b, c, h, w = inputs.shape
    n_tok = b * h * w
    xr = inputs.reshape(b, c, h * w)

    n_steps = n_tok // _TILE
    enc, quantr, loss, perp = pl.pallas_call(
        functools.partial(_vq_tile_kernel, n_tok=n_tok, n_steps=n_steps),
        grid=(n_steps,),
        in_specs=[
            pl.BlockSpec((1, c, h * w), lambda i: (i, 0, 0)),
            pl.BlockSpec((_K, _C), lambda i: (0, 0)),
        ],
        out_specs=[
            pl.BlockSpec((_TILE, _K), lambda i: (i, 0)),
            pl.BlockSpec((1, c, h * w), lambda i: (i, 0, 0)),
            pl.BlockSpec((1, 1), lambda i: (0, 0)),
            pl.BlockSpec((1, 1), lambda i: (0, 0)),
        ],
        out_shape=[
            jax.ShapeDtypeStruct((n_tok, _K), jnp.float32),
            jax.ShapeDtypeStruct((b, c, h * w), jnp.float32),
            jax.ShapeDtypeStruct((1, 1), jnp.float32),
            jax.ShapeDtypeStruct((1, 1), jnp.float32),
        ],
        scratch_shapes=[
            pltpu.VMEM((1, _K), jnp.float32),
            pltpu.VMEM((1, _K), jnp.float32),
            pltpu.VMEM((1, _C), jnp.float32),
        ],
    )(xr, embedding)

    quantized = quantr.reshape(b, c, h, w)
    return (loss.reshape(()), quantized, perp.reshape(()), enc)


# speedup vs baseline: 1.0538x; 1.0538x over previous
import functools

import jax
import jax.numpy as jnp
from jax.experimental import pallas as pl
from jax.experimental.pallas import tpu as pltpu

_K = 1024
_C = 64
_COMMIT = 0.25

_TILE = 1024
_NCHUNK = 4


def _vq_tile_kernel(x_ref, e_ref, enc_ref, quant_ref, loss_ref, perp_ref,
                    esq_acc, colsum_acc, loss_acc, *, n_tok, n_steps):
    i = pl.program_id(0)
    e = e_ref[...]

    @pl.when(i == 0)
    def _init():
        esq_acc[...] = jnp.sum(e * e, axis=1, keepdims=True).reshape(1, _K)
        colsum_acc[...] = jnp.zeros_like(colsum_acc)
        loss_acc[...] = jnp.zeros_like(loss_acc)

    esq = esq_acc[...]
    e_bf = e.astype(jnp.bfloat16)
    cs = _TILE // _NCHUNK
    colsums = []
    losssums = []
    for ci in range(_NCHUNK):
        sl = pl.ds(ci * cs, cs)
        xb = x_ref[0, :, sl]

        xsq = jnp.sum(xb * xb, axis=0, keepdims=True).reshape(cs, 1)
        mm = jax.lax.dot_general(
            xb, e, (((0,), (1,)), ((), ())),
            preferred_element_type=jnp.float32)
        d = (xsq + esq) - 2.0 * mm

        dmin = jnp.min(d, axis=1, keepdims=True)
        iota = jax.lax.broadcasted_iota(jnp.int32, (cs, _K), 1)
        idx = jnp.min(jnp.where(d == dmin, iota, _K), axis=1, keepdims=True)

        onehot = jnp.where(iota == idx, 1.0, 0.0)
        enc_ref[sl, :] = onehot

        quant = jax.lax.dot_general(
            e_bf, onehot.astype(jnp.bfloat16), (((0,), (1,)), ((), ())),
            preferred_element_type=jnp.float32)
        quant_ref[0, :, sl] = xb + (quant - xb)

        colsums.append(jnp.sum(onehot, axis=0, keepdims=True))
        r = quant - xb
        losssums.append(jnp.sum(r * r, axis=1, keepdims=True))

    colsum_acc[...] += sum(colsums)
    loss_acc[...] += sum(losssums).reshape(1, _C)

    @pl.when(i == n_steps - 1)
    def _finalize():
        mse = jnp.sum(loss_acc[...]) / (n_tok * _C)
        loss_ref[...] = jnp.broadcast_to(mse + _COMMIT * mse, (1, 1))
        probs = colsum_acc[...] / n_tok
        ent = jnp.sum(probs * jnp.log(probs + 1e-10))
        perp_ref[...] = jnp.broadcast_to(jnp.exp(-ent), (1, 1))


@jax.jit
def kernel(inputs, embedding):
    b, c, h, w = inputs.shape
    n_tok = b * h * w
    xr = inputs.reshape(b, c, h * w)

    n_steps = n_tok // _TILE
    enc, quantr, loss, perp = pl.pallas_call(
        functools.partial(_vq_tile_kernel, n_tok=n_tok, n_steps=n_steps),
        grid=(n_steps,),
        in_specs=[
            pl.BlockSpec((1, c, h * w), lambda i: (i, 0, 0)),
            pl.BlockSpec((_K, _C), lambda i: (0, 0)),
        ],
        out_specs=[
            pl.BlockSpec((_TILE, _K), lambda i: (i, 0)),
            pl.BlockSpec((1, c, h * w), lambda i: (i, 0, 0)),
            pl.BlockSpec((1, 1), lambda i: (0, 0)),
            pl.BlockSpec((1, 1), lambda i: (0, 0)),
        ],
        out_shape=[
            jax.ShapeDtypeStruct((n_tok, _K), jnp.float32),
            jax.ShapeDtypeStruct((b, c, h * w), jnp.float32),
            jax.ShapeDtypeStruct((1, 1), jnp.float32),
            jax.ShapeDtypeStruct((1, 1), jnp.float32),
        ],
        scratch_shapes=[
            pltpu.VMEM((1, _K), jnp.float32),
            pltpu.VMEM((1, _K), jnp.float32),
            pltpu.VMEM((1, _C), jnp.float32),
        ],
    )(xr, embedding)

    quantized = quantr.reshape(b, c, h, w)
    return (loss.reshape(()), quantized, perp.reshape(()), enc)


# speedup vs baseline: 1.1312x; 1.0735x over previous
import functools

import jax
import jax.numpy as jnp
from jax.experimental import pallas as pl
from jax.experimental.pallas import tpu as pltpu

_K = 1024
_C = 64
_COMMIT = 0.25

_TILE = 512


def _vq_tile_kernel(x_ref, e_ref, enc_ref, quant_ref, loss_ref, perp_ref,
                    colsum_acc, loss_acc, *, n_tok, n_steps):
    i = pl.program_id(0)

    @pl.when(i == 0)
    def _init():
        colsum_acc[...] = jnp.zeros_like(colsum_acc)
        loss_acc[...] = jnp.zeros_like(loss_acc)

    x = x_ref[...]
    e = e_ref[...]

    xsq = jnp.sum(x * x, axis=1, keepdims=True)
    esq = jnp.sum(e * e, axis=1, keepdims=True).reshape(1, _K)
    mm = jnp.dot(x, e.T, preferred_element_type=jnp.float32)
    d = (xsq + esq) - 2.0 * mm

    dmin = jnp.min(d, axis=1, keepdims=True)
    iota = jax.lax.broadcasted_iota(jnp.int32, (_TILE, _K), 1)
    idx = jnp.min(jnp.where(d == dmin, iota, _K), axis=1, keepdims=True)

    onehot = (iota == idx).astype(jnp.float32)
    enc_ref[...] = onehot

    quant = jnp.dot(onehot, e, preferred_element_type=jnp.float32)
    quant_ref[...] = x + (quant - x)

    colsum_acc[...] += jnp.sum(onehot, axis=0, keepdims=True)
    r = quant - x
    loss_acc[...] += jnp.sum(r * r, axis=0, keepdims=True)

    @pl.when(i == n_steps - 1)
    def _finalize():
        mse = jnp.sum(loss_acc[...]) / (n_tok * _C)
        loss_ref[...] = jnp.broadcast_to(mse + _COMMIT * mse, (1, 1))
        probs = colsum_acc[...] / n_tok
        ent = jnp.sum(probs * jnp.log(probs + 1e-10))
        perp_ref[...] = jnp.broadcast_to(jnp.exp(-ent), (1, 1))


@jax.jit
def kernel(inputs, embedding):
    b, c, h, w = inputs.shape
    n_tok = b * h * w
    x = jnp.transpose(inputs, (0, 2, 3, 1)).reshape(n_tok, c)

    n_steps = n_tok // _TILE
    enc, quant, loss, perp = pl.pallas_call(
        functools.partial(_vq_tile_kernel, n_tok=n_tok, n_steps=n_steps),
        grid=(n_steps,),
        in_specs=[
            pl.BlockSpec((_TILE, _C), lambda i: (i, 0)),
            pl.BlockSpec((_K, _C), lambda i: (0, 0)),
        ],
        out_specs=[
            pl.BlockSpec((_TILE, _K), lambda i: (i, 0)),
            pl.BlockSpec((_TILE, _C), lambda i: (i, 0)),
            pl.BlockSpec((1, 1), lambda i: (0, 0)),
            pl.BlockSpec((1, 1), lambda i: (0, 0)),
        ],
        out_shape=[
            jax.ShapeDtypeStruct((n_tok, _K), jnp.float32),
            jax.ShapeDtypeStruct((n_tok, _C), jnp.float32),
            jax.ShapeDtypeStruct((1, 1), jnp.float32),
            jax.ShapeDtypeStruct((1, 1), jnp.float32),
        ],
        scratch_shapes=[
            pltpu.VMEM((1, _K), jnp.float32),
            pltpu.VMEM((1, _C), jnp.float32),
        ],
    )(x, embedding)

    quantized = quant.reshape(b, h, w, c).transpose(0, 3, 1, 2)
    return (loss.reshape(()), quantized, perp.reshape(()), enc)


# speedup vs baseline: 1.3139x; 1.1615x over previous
import functools

import jax
import jax.numpy as jnp
from jax.experimental import pallas as pl
from jax.experimental.pallas import tpu as pltpu

_K = 1024
_C = 64
_COMMIT = 0.25

_TILE = 2048


def _vq_tile_kernel(x_ref, e_ref, enc_ref, quant_ref, loss_ref, perp_ref,
                    colsum_acc, loss_acc, *, n_tok, n_steps):
    i = pl.program_id(0)

    @pl.when(i == 0)
    def _init():
        colsum_acc[...] = jnp.zeros_like(colsum_acc)
        loss_acc[...] = jnp.zeros_like(loss_acc)

    x = x_ref[...]
    e = e_ref[...]

    xsq = jnp.sum(x * x, axis=1, keepdims=True)
    esq = jnp.sum(e * e, axis=1, keepdims=True).reshape(1, _K)
    mm = jnp.dot(x, e.T, preferred_element_type=jnp.float32)
    d = (xsq + esq) - 2.0 * mm

    dmin = jnp.min(d, axis=1, keepdims=True)
    iota = jax.lax.broadcasted_iota(jnp.int32, (_TILE, _K), 1)
    idx = jnp.min(jnp.where(d == dmin, iota, _K), axis=1, keepdims=True)

    onehot = (iota == idx).astype(jnp.float32)
    enc_ref[...] = onehot

    quant = jnp.dot(onehot, e, preferred_element_type=jnp.float32)
    quant_ref[...] = x + (quant - x)

    colsum_acc[...] += jnp.sum(onehot, axis=0, keepdims=True)
    r = quant - x
    loss_acc[...] += jnp.sum(r * r, axis=0, keepdims=True)

    @pl.when(i == n_steps - 1)
    def _finalize():
        mse = jnp.sum(loss_acc[...]) / (n_tok * _C)
        loss_ref[...] = jnp.broadcast_to(mse + _COMMIT * mse, (1, 1))
        probs = colsum_acc[...] / n_tok
        ent = jnp.sum(probs * jnp.log(probs + 1e-10))
        perp_ref[...] = jnp.broadcast_to(jnp.exp(-ent), (1, 1))


@jax.jit
def kernel(inputs, embedding):
    b, c, h, w = inputs.shape
    n_tok = b * h * w
    x = jnp.transpose(inputs, (0, 2, 3, 1)).reshape(n_tok, c)

    n_steps = n_tok // _TILE
    enc, quant, loss, perp = pl.pallas_call(
        functools.partial(_vq_tile_kernel, n_tok=n_tok, n_steps=n_steps),
        grid=(n_steps,),
        in_specs=[
            pl.BlockSpec((_TILE, _C), lambda i: (i, 0)),
            pl.BlockSpec((_K, _C), lambda i: (0, 0)),
        ],
        out_specs=[
            pl.BlockSpec((_TILE, _K), lambda i: (i, 0)),
            pl.BlockSpec((_TILE, _C), lambda i: (i, 0)),
            pl.BlockSpec((1, 1), lambda i: (0, 0)),
            pl.BlockSpec((1, 1), lambda i: (0, 0)),
        ],
        out_shape=[
            jax.ShapeDtypeStruct((n_tok, _K), jnp.float32),
            jax.ShapeDtypeStruct((n_tok, _C), jnp.float32),
            jax.ShapeDtypeStruct((1, 1), jnp.float32),
            jax.ShapeDtypeStruct((1, 1), jnp.float32),
        ],
        scratch_shapes=[
            pltpu.VMEM((1, _K), jnp.float32),
            pltpu.VMEM((1, _C), jnp.float32),
        ],
    )(x, embedding)

    quantized = quant.reshape(b, h, w, c).transpose(0, 3, 1, 2)
    return (loss.reshape(()), quantized, perp.reshape(()), enc)


# speedup vs baseline: 1.5984x; 1.2166x over previous
import functools

import jax
import jax.numpy as jnp
from jax.experimental import pallas as pl
from jax.experimental.pallas import tpu as pltpu

_K = 1024
_C = 64
_COMMIT = 0.25

_TILE = 2048
_NCHUNK = 4


def _vq_tile_kernel(x_ref, e_ref, enc_ref, quant_ref, loss_ref, perp_ref,
                    colsum_acc, loss_acc, *, n_tok, n_steps):
    i = pl.program_id(0)

    @pl.when(i == 0)
    def _init():
        colsum_acc[...] = jnp.zeros_like(colsum_acc)
        loss_acc[...] = jnp.zeros_like(loss_acc)

    e = e_ref[...]
    esq = jnp.sum(e * e, axis=1, keepdims=True).reshape(1, _K)

    cs = _TILE // _NCHUNK
    colsums = []
    losssums = []
    for ci in range(_NCHUNK):
        sl = pl.ds(ci * cs, cs)
        x = x_ref[sl, :]

        xsq = jnp.sum(x * x, axis=1, keepdims=True)
        mm = jnp.dot(x, e.T, preferred_element_type=jnp.float32)
        d = (xsq + esq) - 2.0 * mm

        dmin = jnp.min(d, axis=1, keepdims=True)
        iota = jax.lax.broadcasted_iota(jnp.int32, (cs, _K), 1)
        idx = jnp.min(jnp.where(d == dmin, iota, _K), axis=1, keepdims=True)

        onehot = (iota == idx).astype(jnp.float32)
        enc_ref[sl, :] = onehot

        quant = jnp.dot(onehot, e, preferred_element_type=jnp.float32)
        quant_ref[sl, :] = x + (quant - x)

        colsums.append(jnp.sum(onehot, axis=0, keepdims=True))
        r = quant - x
        losssums.append(jnp.sum(r * r, axis=0, keepdims=True))

    colsum_acc[...] += sum(colsums)
    loss_acc[...] += sum(losssums)

    @pl.when(i == n_steps - 1)
    def _finalize():
        mse = jnp.sum(loss_acc[...]) / (n_tok * _C)
        loss_ref[...] = jnp.broadcast_to(mse + _COMMIT * mse, (1, 1))
        probs = colsum_acc[...] / n_tok
        ent = jnp.sum(probs * jnp.log(probs + 1e-10))
        perp_ref[...] = jnp.broadcast_to(jnp.exp(-ent), (1, 1))


@jax.jit
def kernel(inputs, embedding):
    b, c, h, w = inputs.shape
    n_tok = b * h * w
    x = jnp.transpose(inputs, (0, 2, 3, 1)).reshape(n_tok, c)

    n_steps = n_tok // _TILE
    enc, quant, loss, perp = pl.pallas_call(
        functools.partial(_vq_tile_kernel, n_tok=n_tok, n_steps=n_steps),
        grid=(n_steps,),
        in_specs=[
            pl.BlockSpec((_TILE, _C), lambda i: (i, 0)),
            pl.BlockSpec((_K, _C), lambda i: (0, 0)),
        ],
        out_specs=[
            pl.BlockSpec((_TILE, _K), lambda i: (i, 0)),
            pl.BlockSpec((_TILE, _C), lambda i: (i, 0)),
            pl.BlockSpec((1, 1), lambda i: (0, 0)),
            pl.BlockSpec((1, 1), lambda i: (0, 0)),
        ],
        out_shape=[
            jax.ShapeDtypeStruct((n_tok, _K), jnp.float32),
            jax.ShapeDtypeStruct((n_tok, _C), jnp.float32),
            jax.ShapeDtypeStruct((1, 1), jnp.float32),
            jax.ShapeDtypeStruct((1, 1), jnp.float32),
        ],
        scratch_shapes=[
            pltpu.VMEM((1, _K), jnp.float32),
            pltpu.VMEM((1, _C), jnp.float32),
        ],
    )(x, embedding)

    quantized = quant.reshape(b, h, w, c).transpose(0, 3, 1, 2)
    return (loss.reshape(()), quantized, perp.reshape(()), enc)
